# baseline (speedup 1.0000x reference)
import jax
import jax.numpy as jnp
from jax import lax
from jax.experimental import pallas as pl
from jax.experimental.pallas import tpu as pltpu
from jax.experimental.pallas import tpu_sc as plsc

_B = 16384
_K = 32
_NC = 2
_NS = 16
_NW = _NC * _NS
_BPW = _B // _NW


def _sc_gather(user_table, item_table, user_idx, item_idx):
    mesh = plsc.VectorSubcoreMesh(core_axis_name="c", subcore_axis_name="s")

    @pl.kernel(
        out_type=(jax.ShapeDtypeStruct((_B, _K), jnp.float32),
                  jax.ShapeDtypeStruct((_B, _K), jnp.float32)),
        mesh=mesh,
        scratch_types=[
            pltpu.VMEM((_BPW,), jnp.int32),
            pltpu.VMEM((_BPW,), jnp.int32),
            pltpu.VMEM((_BPW, _K), jnp.float32),
            pltpu.VMEM((_BPW, _K), jnp.float32),
            pltpu.SemaphoreType.DMA,
            pltpu.SemaphoreType.DMA,
        ],
        compiler_params=pltpu.CompilerParams(use_tc_tiling_on_sc=False),
    )
    def gather_kernel(ut_hbm, it_hbm, ui_hbm, ii_hbm, uo_hbm, io_hbm,
                      uidx_v, iidx_v, urows_v, irows_v, usem, isem):
        wid = lax.axis_index("s") * _NC + lax.axis_index("c")
        base = wid * _BPW
        pltpu.sync_copy(ui_hbm.at[pl.ds(base, _BPW)], uidx_v)
        pltpu.sync_copy(ii_hbm.at[pl.ds(base, _BPW)], iidx_v)
        cu = pltpu.async_copy(ut_hbm.at[uidx_v], urows_v, usem)
        ci = pltpu.async_copy(it_hbm.at[iidx_v], irows_v, isem)
        cu.wait()
        ci.wait()
        pltpu.sync_copy(urows_v, uo_hbm.at[pl.ds(base, _BPW)])
        pltpu.sync_copy(irows_v, io_hbm.at[pl.ds(base, _BPW)])

    return gather_kernel(user_table, item_table, user_idx, item_idx)


def _mlp_body(ue_ref, ie_ref, w1u_ref, w1i_ref, b1_ref, w2_ref,
              cvr_ref, ctr_ref, ctcvr_ref):
    ue = ue_ref[...]
    ie = ie_ref[...]
    h = jnp.dot(ue, w1u_ref[...], preferred_element_type=jnp.float32)
    h += jnp.dot(ie, w1i_ref[...], preferred_element_type=jnp.float32)
    h = jnp.maximum(h + b1_ref[...], 0.0)
    ctr = jnp.sum(h * w2_ref[...], axis=1, keepdims=True)
    cvr = jnp.sum(ue * ie, axis=1, keepdims=True)
    cvr_ref[...] = cvr
    ctr_ref[...] = ctr
    ctcvr_ref[...] = jax.nn.sigmoid(ctr) * jax.nn.sigmoid(cvr)


def kernel(x, user_table, item_table, W1, b1, W2):
    xi = x.astype(jnp.int32)
    user_idx = xi[:, 0]
    item_idx = xi[:, 1]

    ue, ie = _sc_gather(user_table, item_table, user_idx, item_idx)

    w1u = W1[:_K]
    w1i = W1[_K:]
    b1r = b1.reshape(1, _K)
    w2r = W2.reshape(1, _K)

    out_t = jax.ShapeDtypeStruct((_B, 1), jnp.float32)
    cvr, ctr, ctcvr = pl.pallas_call(
        _mlp_body,
        out_shape=(out_t, out_t, out_t),
    )(ue, ie, w1u, w1i, b1r, w2r)
    return (cvr, ctr, ctcvr)
